# initial kernel scaffold (unmeasured)
import jax
import jax.numpy as jnp
from jax import lax
from jax.experimental import pallas as pl
from jax.experimental.pallas import tpu as pltpu

S = 2048
S_HALF = 1024
K = 4096
N = 8192
NBLK = 512
NSTEPS = N // NBLK


def kernel(O, Wo):
    o_bf = O.reshape(S, K).astype(jnp.bfloat16)

    def body(o_ref, wo_ref, out_ref, send_buf, recv_buf, send_sems, recv_sems):
        j = pl.program_id(0)
        my_x = lax.axis_index("x")
        my_y = lax.axis_index("y")
        my_z = lax.axis_index("z")
        peer_x = 1 - my_x
        peer = (peer_x, my_y, my_z)

        @pl.when(j == 0)
        def _():
            barrier = pltpu.get_barrier_semaphore()
            pl.semaphore_signal(
                barrier, inc=1, device_id=peer,
                device_id_type=pl.DeviceIdType.MESH,
            )
            pl.semaphore_wait(barrier, 1)

        wo_b = wo_ref[...].astype(jnp.bfloat16)

        o_send = o_ref[pl.ds(peer_x * S_HALF, S_HALF), :]
        p_send = lax.dot_general(
            o_send, wo_b, (((1,), (0,)), ((), ())),
            preferred_element_type=jnp.float32,
        )
        send_buf[0] = p_send.astype(jnp.bfloat16)
        rdma = pltpu.make_async_remote_copy(
            src_ref=send_buf.at[0],
            dst_ref=recv_buf.at[j],
            send_sem=send_sems.at[0],
            recv_sem=recv_sems.at[j],
            device_id=peer,
            device_id_type=pl.DeviceIdType.MESH,
        )
        rdma.start()

        o_keep = o_ref[pl.ds(my_x * S_HALF, S_HALF), :]
        p_keep = lax.dot_general(
            o_keep, wo_b, (((1,), (0,)), ((), ())),
            preferred_element_type=jnp.float32,
        )

        rdma.wait()
        out_ref[...] = p_keep + recv_buf[j].astype(jnp.float32)

    out = pl.pallas_call(
        body,
        grid=(NSTEPS,),
        in_specs=[
            pl.BlockSpec((S, K), lambda j: (0, 0)),
            pl.BlockSpec((K, NBLK), lambda j: (0, j)),
        ],
        out_specs=pl.BlockSpec((S_HALF, NBLK), lambda j: (0, j)),
        out_shape=jax.ShapeDtypeStruct((S_HALF, N), jnp.float32),
        scratch_shapes=[
            pltpu.VMEM((1, S_HALF, NBLK), jnp.bfloat16),
            pltpu.VMEM((NSTEPS, S_HALF, NBLK), jnp.bfloat16),
            pltpu.SemaphoreType.DMA((1,)),
            pltpu.SemaphoreType.DMA((NSTEPS,)),
        ],
        compiler_params=pltpu.CompilerParams(
            collective_id=0,
            dimension_semantics=("arbitrary",),
        ),
    )(o_bf, Wo)
    return out.reshape(1, S_HALF, N)


# baseline (device time: 590993 ns/iter reference)
import jax
import jax.numpy as jnp
from jax import lax
from jax.experimental import pallas as pl
from jax.experimental.pallas import tpu as pltpu

S = 2048
S_HALF = 1024
K = 4096
N = 8192
NBLK = 256
NSTEPS = N // NBLK


def kernel(O, Wo):
    o_bf = O.reshape(S, K).astype(jnp.bfloat16)

    def body(o_ref, wo_ref, out_ref, send_buf, recv_buf, send_sems, recv_sems):
        j = pl.program_id(0)
        my_x = lax.axis_index("x")
        my_y = lax.axis_index("y")
        my_z = lax.axis_index("z")
        peer_x = 1 - my_x
        peer = (peer_x, my_y, my_z)

        @pl.when(j == 0)
        def _():
            barrier = pltpu.get_barrier_semaphore()
            pl.semaphore_signal(
                barrier, inc=1, device_id=peer,
                device_id_type=pl.DeviceIdType.MESH,
            )
            pl.semaphore_wait(barrier, 1)

        wo_b = wo_ref[...].astype(jnp.bfloat16)

        o_send = o_ref[pl.ds(peer_x * S_HALF, S_HALF), :]
        p_send = lax.dot_general(
            o_send, wo_b, (((1,), (0,)), ((), ())),
            preferred_element_type=jnp.float32,
        )
        send_buf[0] = p_send.astype(jnp.bfloat16)
        rdma = pltpu.make_async_remote_copy(
            src_ref=send_buf.at[0],
            dst_ref=recv_buf.at[j],
            send_sem=send_sems.at[0],
            recv_sem=recv_sems.at[j],
            device_id=peer,
            device_id_type=pl.DeviceIdType.MESH,
        )
        rdma.start()

        o_keep = o_ref[pl.ds(my_x * S_HALF, S_HALF), :]
        p_keep = lax.dot_general(
            o_keep, wo_b, (((1,), (0,)), ((), ())),
            preferred_element_type=jnp.float32,
        )

        rdma.wait()
        out_ref[...] = p_keep + recv_buf[j].astype(jnp.float32)

    out = pl.pallas_call(
        body,
        grid=(NSTEPS,),
        in_specs=[
            pl.BlockSpec((S, K), lambda j: (0, 0)),
            pl.BlockSpec((K, NBLK), lambda j: (0, j)),
        ],
        out_specs=pl.BlockSpec((S_HALF, NBLK), lambda j: (0, j)),
        out_shape=jax.ShapeDtypeStruct((S_HALF, N), jnp.float32),
        scratch_shapes=[
            pltpu.VMEM((1, S_HALF, NBLK), jnp.bfloat16),
            pltpu.VMEM((NSTEPS, S_HALF, NBLK), jnp.bfloat16),
            pltpu.SemaphoreType.DMA((1,)),
            pltpu.SemaphoreType.DMA((NSTEPS,)),
        ],
        compiler_params=pltpu.CompilerParams(
            collective_id=0,
            dimension_semantics=("arbitrary",),
            vmem_limit_bytes=60 * 1024 * 1024,
        ),
    )(o_bf, Wo)
    return out.reshape(1, S_HALF, N)


# device time: 355962 ns/iter; 1.6603x vs baseline; 1.6603x over previous
import jax
import jax.numpy as jnp
from jax import lax
from jax.experimental import pallas as pl
from jax.experimental.pallas import tpu as pltpu

S = 2048
S_HALF = 1024
K = 4096
N = 8192
NBLK = 256
NSTEPS = N // NBLK


def kernel(O, Wo):
    o_bf = O.reshape(S, K).astype(jnp.bfloat16)

    def body(o_ref, wo_ref, out_ref,
             send_buf, recv_buf, keep_buf,
             send_sems, recv_sems):
        j = pl.program_id(0)
        my_x = lax.axis_index("x")
        my_y = lax.axis_index("y")
        my_z = lax.axis_index("z")
        peer_x = 1 - my_x
        peer = (peer_x, my_y, my_z)

        def send_rdma(k):
            return pltpu.make_async_remote_copy(
                src_ref=send_buf.at[k % 2],
                dst_ref=recv_buf.at[k],
                send_sem=send_sems.at[k % 2],
                recv_sem=recv_sems.at[k],
                device_id=peer,
                device_id_type=pl.DeviceIdType.MESH,
            )

        @pl.when(j == 0)
        def _():
            barrier = pltpu.get_barrier_semaphore()
            pl.semaphore_signal(
                barrier, inc=1, device_id=peer,
                device_id_type=pl.DeviceIdType.MESH,
            )
            pl.semaphore_wait(barrier, 1)

        @pl.when(j < NSTEPS)
        def _():
            wo_b = wo_ref[...].astype(jnp.bfloat16)

            o_send = o_ref[pl.ds(peer_x * S_HALF, S_HALF), :]
            p_send = lax.dot_general(
                o_send, wo_b, (((1,), (0,)), ((), ())),
                preferred_element_type=jnp.float32,
            )

            @pl.when(j >= 2)
            def _():
                send_rdma(j - 2).wait_send()

            send_buf[j % 2] = p_send.astype(jnp.bfloat16)
            send_rdma(j).start()

            o_keep = o_ref[pl.ds(my_x * S_HALF, S_HALF), :]
            keep_buf[j % 2] = lax.dot_general(
                o_keep, wo_b, (((1,), (0,)), ((), ())),
                preferred_element_type=jnp.float32,
            )

        @pl.when(j >= 1)
        def _():
            i = j - 1
            send_rdma(i).wait_recv()
            out_ref[...] = keep_buf[i % 2] + recv_buf[i].astype(jnp.float32)

        @pl.when(j == NSTEPS)
        def _():
            send_rdma(NSTEPS - 2).wait_send()
            send_rdma(NSTEPS - 1).wait_send()

    out = pl.pallas_call(
        body,
        grid=(NSTEPS + 1,),
        in_specs=[
            pl.BlockSpec((S, K), lambda j: (0, 0)),
            pl.BlockSpec((K, NBLK),
                         lambda j: (0, jnp.minimum(j, NSTEPS - 1))),
        ],
        out_specs=pl.BlockSpec((S_HALF, NBLK),
                               lambda j: (0, jnp.maximum(j - 1, 0))),
        out_shape=jax.ShapeDtypeStruct((S_HALF, N), jnp.float32),
        scratch_shapes=[
            pltpu.VMEM((2, S_HALF, NBLK), jnp.bfloat16),
            pltpu.VMEM((NSTEPS, S_HALF, NBLK), jnp.bfloat16),
            pltpu.VMEM((2, S_HALF, NBLK), jnp.float32),
            pltpu.SemaphoreType.DMA((2,)),
            pltpu.SemaphoreType.DMA((NSTEPS,)),
        ],
        compiler_params=pltpu.CompilerParams(
            collective_id=0,
            dimension_semantics=("arbitrary",),
            vmem_limit_bytes=60 * 1024 * 1024,
        ),
    )(o_bf, Wo)
    return out.reshape(1, S_HALF, N)


# device time: 247207 ns/iter; 2.3907x vs baseline; 1.4399x over previous
import jax
import jax.numpy as jnp
from jax import lax
from jax.experimental import pallas as pl
from jax.experimental.pallas import tpu as pltpu

S = 2048
S_HALF = 1024
K = 4096
N = 8192
NBLK = 512
NSTEPS = N // NBLK
NSLOTS = 4


def kernel(O, Wo):
    o_bf = O.reshape(S, K).astype(jnp.bfloat16)

    def body(o_ref, wo_ref, out_ref,
             send_buf, recv_buf, keep_buf,
             send_sems, recv_sems, credit_sem):
        j = pl.program_id(0)
        my_x = lax.axis_index("x")
        my_y = lax.axis_index("y")
        my_z = lax.axis_index("z")
        peer_x = 1 - my_x
        peer = (peer_x, my_y, my_z)

        def send_rdma(k):
            return pltpu.make_async_remote_copy(
                src_ref=send_buf.at[k % 2],
                dst_ref=recv_buf.at[k % NSLOTS],
                send_sem=send_sems.at[k % 2],
                recv_sem=recv_sems.at[k % NSLOTS],
                device_id=peer,
                device_id_type=pl.DeviceIdType.MESH,
            )

        @pl.when(j == 0)
        def _():
            barrier = pltpu.get_barrier_semaphore()
            pl.semaphore_signal(
                barrier, inc=1, device_id=peer,
                device_id_type=pl.DeviceIdType.MESH,
            )
            pl.semaphore_wait(barrier, 1)

        @pl.when(j < NSTEPS)
        def _():
            wo_b = wo_ref[...].astype(jnp.bfloat16)

            o_send = o_ref[pl.ds(peer_x * S_HALF, S_HALF), :]
            p_send = lax.dot_general(
                o_send, wo_b, (((1,), (0,)), ((), ())),
                preferred_element_type=jnp.float32,
            )

            @pl.when(j >= 2)
            def _():
                send_rdma(j - 2).wait_send()

            send_buf[j % 2] = p_send.astype(jnp.bfloat16)

            @pl.when(j >= NSLOTS)
            def _():
                pl.semaphore_wait(credit_sem, 1)

            send_rdma(j).start()

            o_keep = o_ref[pl.ds(my_x * S_HALF, S_HALF), :]
            keep_buf[j % 2] = lax.dot_general(
                o_keep, wo_b, (((1,), (0,)), ((), ())),
                preferred_element_type=jnp.float32,
            )

        @pl.when(j >= 1)
        def _():
            i = j - 1
            send_rdma(i).wait_recv()
            out_ref[...] = keep_buf[i % 2] + recv_buf[i % NSLOTS].astype(
                jnp.float32)
            pl.semaphore_signal(
                credit_sem, inc=1, device_id=peer,
                device_id_type=pl.DeviceIdType.MESH,
            )

        @pl.when(j == NSTEPS)
        def _():
            send_rdma(NSTEPS - 2).wait_send()
            send_rdma(NSTEPS - 1).wait_send()
            pl.semaphore_wait(credit_sem, NSLOTS)

    out = pl.pallas_call(
        body,
        grid=(NSTEPS + 1,),
        in_specs=[
            pl.BlockSpec((S, K), lambda j: (0, 0)),
            pl.BlockSpec((K, NBLK),
                         lambda j: (0, jnp.minimum(j, NSTEPS - 1))),
        ],
        out_specs=pl.BlockSpec((S_HALF, NBLK),
                               lambda j: (0, jnp.maximum(j - 1, 0))),
        out_shape=jax.ShapeDtypeStruct((S_HALF, N), jnp.float32),
        scratch_shapes=[
            pltpu.VMEM((2, S_HALF, NBLK), jnp.bfloat16),
            pltpu.VMEM((NSLOTS, S_HALF, NBLK), jnp.bfloat16),
            pltpu.VMEM((2, S_HALF, NBLK), jnp.float32),
            pltpu.SemaphoreType.DMA((2,)),
            pltpu.SemaphoreType.DMA((NSLOTS,)),
            pltpu.SemaphoreType.REGULAR,
        ],
        compiler_params=pltpu.CompilerParams(
            collective_id=0,
            dimension_semantics=("arbitrary",),
            vmem_limit_bytes=60 * 1024 * 1024,
        ),
    )(o_bf, Wo)
    return out.reshape(1, S_HALF, N)
